# baseline (speedup 1.0000x reference)
import functools

import jax
import jax.numpy as jnp
from jax import lax
from jax.experimental import pallas as pl
from jax.experimental.pallas import tpu as pltpu
from jax.experimental.pallas import tpu_sc as plsc

N = 16384
K = 100000
D = 64
KH = 50176
DW = 2 * D
L = 16
NC = 2
NS = 16
NW = NC * NS
N_PER = N // NW
CHUNK = 128
NCHUNK = N_PER // CHUNK
GPC = CHUNK // L

_MAGIC = 0x5F3759DF


def _rsqrt_newton(x):
    i = plsc.bitcast(x, jnp.int32)
    i = _MAGIC - lax.shift_right_logical(i, 1)
    y = plsc.bitcast(i, jnp.float32)
    xh = x * 0.5
    for _ in range(3):
        y = y * (1.5 - xh * y * y)
    return y


def _make_sc_kernel():
    mesh = plsc.VectorSubcoreMesh(core_axis_name="c", subcore_axis_name="s")

    @functools.partial(
        pl.kernel,
        mesh=mesh,
        out_type=jax.ShapeDtypeStruct((NW, L), jnp.float32),
        compiler_params=pltpu.CompilerParams(
            needs_layout_passes=False, use_tc_tiling_on_sc=False),
        scratch_types=[
            pltpu.VMEM((NCHUNK, CHUNK), jnp.int32),
            pltpu.VMEM((NCHUNK, CHUNK), jnp.int32),
            pltpu.VMEM((NCHUNK, CHUNK), jnp.int32),
            pltpu.VMEM((NCHUNK, CHUNK), jnp.int32),
            pltpu.VMEM((2, CHUNK, DW), jnp.float32),
            pltpu.VMEM((2, CHUNK, DW), jnp.float32),
            pltpu.VMEM((1, L), jnp.float32),
            pltpu.SemaphoreType.DMA,
            [pltpu.SemaphoreType.DMA] * NCHUNK,
        ],
    )
    def vloss(pred_hbm, tgt_hbm, table_hbm, out_hbm,
              idxp, idxt, rowp, rowt, pbuf, tbuf, accv, isem, gsems):
        wid = lax.axis_index("s") * NC + lax.axis_index("c")
        base = wid * N_PER

        ih = []
        for j in range(NCHUNK):
            ih.append(pltpu.async_copy(
                pred_hbm.at[pl.ds(base + j * CHUNK, CHUNK)], idxp.at[j], isem))
            ih.append(pltpu.async_copy(
                tgt_hbm.at[pl.ds(base + j * CHUNK, CHUNK)], idxt.at[j], isem))
        for h in ih:
            h.wait()

        for j in range(NCHUNK):
            for g in range(GPC):
                sl = pl.ds(g * L, L)
                vp = idxp[j, sl]
                vt = idxt[j, sl]
                rowp[j, sl] = vp - jnp.where(vp >= KH, KH, 0)
                rowt[j, sl] = vt - jnp.where(vt >= KH, KH, 0)

        def fire(j):
            b = j % 2
            return (
                pltpu.async_copy(table_hbm.at[rowp.at[j]], pbuf.at[b],
                                 gsems[j]),
                pltpu.async_copy(table_hbm.at[rowt.at[j]], tbuf.at[b],
                                 gsems[j]),
            )

        inflight = {0: fire(0), 1: fire(1)}

        lane = lax.iota(jnp.int32, L)
        zero = jnp.zeros((L,), jnp.float32)

        acc = zero
        for j in range(NCHUNK):
            for h in inflight[j]:
                h.wait()
            b = j % 2
            pb = pbuf.at[b]
            tb = tbuf.at[b]

            def body(g, a, j=j, pb=pb, tb=tb):
                sl = pl.ds(g * L, L)
                offp = jnp.where(idxp[j, sl] >= KH, D, 0)
                offt = jnp.where(idxt[j, sl] >= KH, D, 0)
                rows = g * L + lane
                s0 = zero
                s1 = zero
                s2 = zero
                s3 = zero
                cols = lane
                for c in range(D):
                    dp = plsc.load_gather(pb, [rows, offp + cols])
                    dt = plsc.load_gather(tb, [rows, offt + cols])
                    cols = (cols + 1) & (D - 1)
                    dd = dp - dt
                    if c % 4 == 0:
                        s0 = s0 + dd * dd
                    elif c % 4 == 1:
                        s1 = s1 + dd * dd
                    elif c % 4 == 2:
                        s2 = s2 + dd * dd
                    else:
                        s3 = s3 + dd * dd
                s = (s0 + s1) + (s2 + s3)
                x = jnp.maximum(s, 1e-30)
                return a + x * _rsqrt_newton(x)

            acc = lax.fori_loop(0, GPC, body, acc)
            if j + 2 < NCHUNK:
                inflight[j + 2] = fire(j + 2)
        accv[0, :] = acc
        pltpu.sync_copy(accv, out_hbm.at[pl.ds(wid, 1)])

    return vloss


_vloss = _make_sc_kernel()

TBLK = 12544
_TGRID = KH // TBLK


def _tr_body(xl_ref, xr_ref, o_ref):
    r = lax.broadcasted_iota(jnp.int32, (D, D), 0)
    c = lax.broadcasted_iota(jnp.int32, (D, D), 1)
    eye = jnp.where(r == c, 1.0, 0.0).astype(jnp.float32)
    dims = (((0,), (0,)), ((), ()))
    o_ref[:, 0:D] = lax.dot_general(
        xl_ref[...], eye, dims, preferred_element_type=jnp.float32)
    o_ref[:, D:DW] = lax.dot_general(
        xr_ref[...], eye, dims, preferred_element_type=jnp.float32)


_tc_transpose = pl.pallas_call(
    _tr_body,
    grid=(_TGRID,),
    in_specs=[
        pl.BlockSpec((D, TBLK), lambda j: (0, j)),
        pl.BlockSpec((D, TBLK), lambda j: (0, j + _TGRID)),
    ],
    out_specs=pl.BlockSpec((TBLK, DW), lambda j: (j, 0)),
    out_shape=jax.ShapeDtypeStruct((KH, DW), jnp.float32),
)


def kernel(predicted, targets, centroids):
    ct = jnp.transpose(centroids)
    table = _tc_transpose(ct, ct)
    parts = _vloss(predicted, targets, table)
    return jnp.sum(parts) * jnp.float32(1.0 / N)


# speedup vs baseline: 1.0373x; 1.0373x over previous
import functools

import jax
import jax.numpy as jnp
from jax import lax
from jax.experimental import pallas as pl
from jax.experimental.pallas import tpu as pltpu
from jax.experimental.pallas import tpu_sc as plsc

N = 16384
K = 100000
D = 64
KH = 50176
DW = 2 * D
L = 16
NC = 2
NS = 16
NW = NC * NS
N_PER = N // NW
CHUNK = 128
NCHUNK = N_PER // CHUNK
GPC = CHUNK // L

_MAGIC = 0x5F3759DF


def _rsqrt_newton(x):
    i = plsc.bitcast(x, jnp.int32)
    i = _MAGIC - lax.shift_right_logical(i, 1)
    y = plsc.bitcast(i, jnp.float32)
    xh = x * 0.5
    for _ in range(3):
        y = y * (1.5 - xh * y * y)
    return y


def _make_sc_kernel():
    mesh = plsc.VectorSubcoreMesh(core_axis_name="c", subcore_axis_name="s")

    @functools.partial(
        pl.kernel,
        mesh=mesh,
        out_type=jax.ShapeDtypeStruct((NW, L), jnp.float32),
        compiler_params=pltpu.CompilerParams(
            needs_layout_passes=False, use_tc_tiling_on_sc=False),
        scratch_types=[
            pltpu.VMEM((NCHUNK, CHUNK), jnp.int32),
            pltpu.VMEM((NCHUNK, CHUNK), jnp.int32),
            pltpu.VMEM((NCHUNK, CHUNK), jnp.int32),
            pltpu.VMEM((NCHUNK, CHUNK), jnp.int32),
            pltpu.VMEM((3, CHUNK, DW), jnp.float32),
            pltpu.VMEM((3, CHUNK, DW), jnp.float32),
            pltpu.VMEM((1, L), jnp.float32),
            pltpu.SemaphoreType.DMA,
            [pltpu.SemaphoreType.DMA] * NCHUNK,
        ],
    )
    def vloss(pred_hbm, tgt_hbm, table_hbm, out_hbm,
              idxp, idxt, rowp, rowt, pbuf, tbuf, accv, isem, gsems):
        wid = lax.axis_index("s") * NC + lax.axis_index("c")
        base = wid * N_PER

        ih = []
        for j in range(NCHUNK):
            ih.append(pltpu.async_copy(
                pred_hbm.at[pl.ds(base + j * CHUNK, CHUNK)], idxp.at[j], isem))
            ih.append(pltpu.async_copy(
                tgt_hbm.at[pl.ds(base + j * CHUNK, CHUNK)], idxt.at[j], isem))
        for h in ih:
            h.wait()

        for j in range(NCHUNK):
            for g in range(GPC):
                sl = pl.ds(g * L, L)
                vp = idxp[j, sl]
                vt = idxt[j, sl]
                rowp[j, sl] = vp - jnp.where(vp >= KH, KH, 0)
                rowt[j, sl] = vt - jnp.where(vt >= KH, KH, 0)

        def fire(j):
            b = j % 3
            return (
                pltpu.async_copy(table_hbm.at[rowp.at[j]], pbuf.at[b],
                                 gsems[j]),
                pltpu.async_copy(table_hbm.at[rowt.at[j]], tbuf.at[b],
                                 gsems[j]),
            )

        inflight = {j: fire(j) for j in range(3)}

        lane = lax.iota(jnp.int32, L)
        zero = jnp.zeros((L,), jnp.float32)

        acc = zero
        for j in range(NCHUNK):
            for h in inflight[j]:
                h.wait()
            pb = pbuf.at[j % 3]
            tb = tbuf.at[j % 3]

            def body(g, a, j=j, pb=pb, tb=tb):
                sl = pl.ds(g * L, L)
                offp = jnp.where(idxp[j, sl] >= KH, D, 0)
                offt = jnp.where(idxt[j, sl] >= KH, D, 0)
                rows = g * L + lane
                s0 = zero
                s1 = zero
                s2 = zero
                s3 = zero
                cols = lane
                for c in range(D):
                    dp = plsc.load_gather(pb, [rows, offp + cols])
                    dt = plsc.load_gather(tb, [rows, offt + cols])
                    cols = (cols + 1) & (D - 1)
                    dd = dp - dt
                    if c % 4 == 0:
                        s0 = s0 + dd * dd
                    elif c % 4 == 1:
                        s1 = s1 + dd * dd
                    elif c % 4 == 2:
                        s2 = s2 + dd * dd
                    else:
                        s3 = s3 + dd * dd
                s = (s0 + s1) + (s2 + s3)
                x = jnp.maximum(s, 1e-30)
                return a + x * _rsqrt_newton(x)

            acc = lax.fori_loop(0, GPC, body, acc)
            if j + 3 < NCHUNK:
                inflight[j + 3] = fire(j + 3)
        accv[0, :] = acc
        pltpu.sync_copy(accv, out_hbm.at[pl.ds(wid, 1)])

    return vloss


_vloss = _make_sc_kernel()

TBLK = 7168
_TGRID = KH // TBLK


def _tr_body(xl_ref, xr_ref, o_ref):
    r = lax.broadcasted_iota(jnp.int32, (D, DW), 0)
    c = lax.broadcasted_iota(jnp.int32, (D, DW), 1)
    eye_l = jnp.where(r == c, 1.0, 0.0).astype(jnp.float32)
    eye_r = jnp.where(r + D == c, 1.0, 0.0).astype(jnp.float32)
    dims = (((0,), (0,)), ((), ()))
    o_ref[...] = (
        lax.dot_general(xl_ref[...], eye_l, dims,
                        preferred_element_type=jnp.float32)
        + lax.dot_general(xr_ref[...], eye_r, dims,
                          preferred_element_type=jnp.float32))


_tc_transpose = pl.pallas_call(
    _tr_body,
    grid=(_TGRID,),
    in_specs=[
        pl.BlockSpec((D, TBLK), lambda j: (0, j)),
        pl.BlockSpec((D, TBLK), lambda j: (0, j + _TGRID)),
    ],
    out_specs=pl.BlockSpec((TBLK, DW), lambda j: (j, 0)),
    out_shape=jax.ShapeDtypeStruct((KH, DW), jnp.float32),
)


def kernel(predicted, targets, centroids):
    ct = jnp.transpose(centroids)
    table = _tc_transpose(ct, ct)
    parts = _vloss(predicted, targets, table)
    return jnp.sum(parts) * jnp.float32(1.0 / N)


# speedup vs baseline: 1.0443x; 1.0067x over previous
import functools

import jax
import jax.numpy as jnp
from jax import lax
from jax.experimental import pallas as pl
from jax.experimental.pallas import tpu as pltpu
from jax.experimental.pallas import tpu_sc as plsc

N = 16384
K = 100000
D = 64
KH = 50176
DW = 2 * D
L = 16
NC = 2
NS = 16
NW = NC * NS
N_PER = N // NW
CHUNK = 128
NCHUNK = N_PER // CHUNK
GPC = CHUNK // L

_MAGIC = 0x5F3759DF


def _rsqrt_newton(x):
    i = plsc.bitcast(x, jnp.int32)
    i = _MAGIC - lax.shift_right_logical(i, 1)
    y = plsc.bitcast(i, jnp.float32)
    xh = x * 0.5
    for _ in range(3):
        y = y * (1.5 - xh * y * y)
    return y


def _make_sc_kernel():
    mesh = plsc.VectorSubcoreMesh(core_axis_name="c", subcore_axis_name="s")

    @functools.partial(
        pl.kernel,
        mesh=mesh,
        out_type=jax.ShapeDtypeStruct((NW, L), jnp.float32),
        compiler_params=pltpu.CompilerParams(
            needs_layout_passes=False, use_tc_tiling_on_sc=False),
        scratch_types=[
            pltpu.VMEM((NCHUNK, CHUNK), jnp.int32),
            pltpu.VMEM((NCHUNK, CHUNK), jnp.int32),
            pltpu.VMEM((NCHUNK, CHUNK), jnp.int32),
            pltpu.VMEM((NCHUNK, CHUNK), jnp.int32),
            pltpu.VMEM((3, CHUNK, D), jnp.float32),
            pltpu.VMEM((3, CHUNK, D), jnp.float32),
            pltpu.VMEM((1, L), jnp.float32),
            pltpu.SemaphoreType.DMA,
            [pltpu.SemaphoreType.DMA] * NCHUNK,
        ],
    )
    def vloss(pred_hbm, tgt_hbm, table_hbm, out_hbm,
              idxp, idxt, rowp, rowt, pbuf, tbuf, accv, isem, gsems):
        wid = lax.axis_index("s") * NC + lax.axis_index("c")
        base = wid * N_PER

        ih = []
        for j in range(NCHUNK):
            ih.append(pltpu.async_copy(
                pred_hbm.at[pl.ds(base + j * CHUNK, CHUNK)], idxp.at[j], isem))
            ih.append(pltpu.async_copy(
                tgt_hbm.at[pl.ds(base + j * CHUNK, CHUNK)], idxt.at[j], isem))
        for h in ih:
            h.wait()

        for j in range(NCHUNK):
            for g in range(GPC):
                sl = pl.ds(g * L, L)
                vp = idxp[j, sl]
                vt = idxt[j, sl]
                rowp[j, sl] = 2 * vp - jnp.where(vp >= KH, 2 * KH - 1, 0)
                rowt[j, sl] = 2 * vt - jnp.where(vt >= KH, 2 * KH - 1, 0)

        def fire(j):
            b = j % 3
            return (
                pltpu.async_copy(table_hbm.at[rowp.at[j]], pbuf.at[b],
                                 gsems[j]),
                pltpu.async_copy(table_hbm.at[rowt.at[j]], tbuf.at[b],
                                 gsems[j]),
            )

        inflight = {j: fire(j) for j in range(3)}

        lane = lax.iota(jnp.int32, L)
        zero = jnp.zeros((L,), jnp.float32)

        acc = zero
        for j in range(NCHUNK):
            for h in inflight[j]:
                h.wait()
            pb = pbuf.at[j % 3]
            tb = tbuf.at[j % 3]

            def body(g, a, j=j, pb=pb, tb=tb):
                rows = g * L + lane
                s0 = zero
                s1 = zero
                s2 = zero
                s3 = zero
                cols = lane
                for c in range(D):
                    dp = plsc.load_gather(pb, [rows, cols])
                    dt = plsc.load_gather(tb, [rows, cols])
                    cols = (cols + 1) & (D - 1)
                    dd = dp - dt
                    if c % 4 == 0:
                        s0 = s0 + dd * dd
                    elif c % 4 == 1:
                        s1 = s1 + dd * dd
                    elif c % 4 == 2:
                        s2 = s2 + dd * dd
                    else:
                        s3 = s3 + dd * dd
                s = (s0 + s1) + (s2 + s3)
                x = jnp.maximum(s, 1e-30)
                return a + x * _rsqrt_newton(x)

            acc = lax.fori_loop(0, GPC, body, acc)
            if j + 3 < NCHUNK:
                inflight[j + 3] = fire(j + 3)
        accv[0, :] = acc
        pltpu.sync_copy(accv, out_hbm.at[pl.ds(wid, 1)])

    return vloss


_vloss = _make_sc_kernel()

TBLK = 7168
_TGRID = KH // TBLK


def _tr_body(xl_ref, xr_ref, o_ref):
    r = lax.broadcasted_iota(jnp.int32, (D, DW), 0)
    c = lax.broadcasted_iota(jnp.int32, (D, DW), 1)
    eye_l = jnp.where(r == c, 1.0, 0.0).astype(jnp.float32)
    eye_r = jnp.where(r + D == c, 1.0, 0.0).astype(jnp.float32)
    dims = (((0,), (0,)), ((), ()))
    o_ref[...] = (
        lax.dot_general(xl_ref[...], eye_l, dims,
                        preferred_element_type=jnp.float32)
        + lax.dot_general(xr_ref[...], eye_r, dims,
                          preferred_element_type=jnp.float32))


_tc_transpose = pl.pallas_call(
    _tr_body,
    grid=(_TGRID,),
    in_specs=[
        pl.BlockSpec((D, TBLK), lambda j: (0, j)),
        pl.BlockSpec((D, TBLK), lambda j: (0, j + _TGRID)),
    ],
    out_specs=pl.BlockSpec((TBLK, DW), lambda j: (j, 0)),
    out_shape=jax.ShapeDtypeStruct((KH, DW), jnp.float32),
)


def kernel(predicted, targets, centroids):
    ct = jnp.transpose(centroids)
    table = _tc_transpose(ct, ct)
    table = jnp.reshape(table, (2 * KH, D))
    parts = _vloss(predicted, targets, table)
    return jnp.sum(parts) * jnp.float32(1.0 / N)
